# baseline (speedup 1.0000x reference)
import functools

import jax
import jax.numpy as jnp
from jax import lax
from jax.experimental import pallas as pl
from jax.experimental.pallas import tpu as pltpu
from jax.experimental.pallas import tpu_sc as plsc

NC = 2
NS = 16
NW = NC * NS


def _sc_aggregate(x_pad, edges3, ident_in,
                  n_pad, d, k, nchunk):
    half = n_pad // NC
    acc_rows = half + 128
    nz = half // NS
    nw_rows = half // NS
    cr = half // 128
    hr = ((cr + 1 + 7) // 8) * 8
    mesh = plsc.VectorSubcoreMesh(core_axis_name="c", subcore_axis_name="s")

    @functools.partial(
        pl.kernel,
        mesh=mesh,
        compiler_params=pltpu.CompilerParams(needs_layout_passes=False),
        out_type=[
            jax.ShapeDtypeStruct((n_pad, d), jnp.float32),
            jax.ShapeDtypeStruct((NC, cr, 128), jnp.float32),
        ],
        scratch_types=[
            pltpu.VMEM((nchunk, k), jnp.int32),
            pltpu.VMEM((nchunk, k), jnp.int32),
            pltpu.VMEM((k, d), jnp.float32),
            pltpu.VMEM((k, d), jnp.float32),
            pltpu.VMEM((8, d), jnp.float32),
            pltpu.VMEM((hr, 128), jnp.float32),
            pltpu.VMEM((1, hr), jnp.int32),
            pltpu.VMEM_SHARED((acc_rows, d), jnp.float32),
            pltpu.VMEM_SHARED((hr, 128), jnp.float32),
            pltpu.SemaphoreType.DMA,
            pltpu.SemaphoreType.DMA,
            pltpu.SemaphoreType.DMA,
            pltpu.SemaphoreType.DMA,
        ],
    )
    def body(x_hbm, edges_hbm, ident_hbm,
             sum_hbm, cnt_hbm,
             src_v, dst_v, rows_a, rows_b, zrow_v, hist_v, ident_v,
             acc_sh, cnt_sh, sg_a, sg_b, ss_a, ss_b):
        c = lax.axis_index("c")
        s = lax.axis_index("s")

        zv = jnp.zeros((16,), jnp.float32)

        for r in range(8):
            for cc in range(d // 16):
                zrow_v[r, pl.ds(cc * 16, 16)] = zv

        def zero_hist(i, carry):
            for cc in range(128 // 16):
                hist_v[i, pl.ds(cc * 16, 16)] = zv
            return carry

        lax.fori_loop(0, hr, zero_hist, 0)

        @pl.when(s < hr // 8)
        def _():
            pltpu.sync_copy(zrow_v, cnt_sh.at[pl.ds(s * 8, 8)])

        pltpu.sync_copy(ident_hbm, ident_v)

        z0 = s * nz

        def zero_body(i, carry):
            pltpu.sync_copy(zrow_v, acc_sh.at[pl.ds(z0 + i * 8, 8)])
            return carry

        lax.fori_loop(0, nz // 8, zero_body, 0)

        pltpu.sync_copy(zrow_v, acc_sh.at[pl.ds(half + s * 8, 8)])

        pltpu.sync_copy(edges_hbm.at[s], src_v)

        lo = c * half
        hi = lo + half
        dump0 = half + lax.shift_left(lax.iota(jnp.int32, 16), 3)
        m14 = jnp.full((16,), (1 << 14) - 1, jnp.int32)

        def xform_body(r, carry):
            dump = dump0 + lax.bitwise_and(r, 7)
            for q in range(k // 16):
                pe = src_v[r, pl.ds(q * 16, 16)]
                v = lax.shift_right_logical(pe, 14)
                ok = (v >= lo) & (v < hi)
                src_v[r, pl.ds(q * 16, 16)] = lax.bitwise_and(pe, m14)
                dst_v[r, pl.ds(q * 16, 16)] = jnp.where(ok, v - lo, dump)
            return carry

        lax.fori_loop(0, nchunk, xform_body, 0)

        plsc.subcore_barrier()

        def fire_gather(j, rows, sg):
            pltpu.async_copy(x_hbm.at[src_v.at[j]], rows, sg)

        def wait_gather(j, rows, sg):
            pltpu.make_async_copy(x_hbm.at[src_v.at[j]], rows, sg).wait()

        def fire_scatter(j, rows, ss):
            pltpu.async_copy(rows, acc_sh.at[dst_v.at[j]], ss, add=True)

        def wait_scatter(j, rows, ss):
            pltpu.make_async_copy(rows, acc_sh.at[dst_v.at[j]], ss).wait()

        def hist_update(j):
            for q in range(k // 16):
                idx = dst_v[j, pl.ds(q * 16, 16)]
                occ, last = plsc.scan_count(idx)
                row = lax.shift_right_logical(idx, 7)
                col = lax.bitwise_and(idx, 127)
                old = plsc.load_gather(hist_v, [row, col])
                plsc.store_scatter(hist_v, [row, col],
                                   old + occ.astype(jnp.float32), mask=last)

        fire_gather(0, rows_a, sg_a)
        wait_gather(0, rows_a, sg_a)
        fire_scatter(0, rows_a, ss_a)
        fire_gather(1, rows_b, sg_b)
        hist_update(0)

        def pair_body(g, carry):
            jb = 1 + 2 * g
            wait_gather(jb, rows_b, sg_b)
            fire_scatter(jb, rows_b, ss_b)
            wait_scatter(jb - 1, rows_a, ss_a)
            fire_gather(jb + 1, rows_a, sg_a)
            hist_update(jb)

            ja = jb + 1
            wait_gather(ja, rows_a, sg_a)
            fire_scatter(ja, rows_a, ss_a)
            wait_scatter(ja - 1, rows_b, ss_b)
            fire_gather(ja + 1, rows_b, sg_b)
            hist_update(ja)
            return carry

        lax.fori_loop(0, (nchunk - 2) // 2, pair_body, 0)

        jt = nchunk - 1
        wait_gather(jt, rows_b, sg_b)
        fire_scatter(jt, rows_b, ss_b)
        wait_scatter(jt - 1, rows_a, ss_a)
        hist_update(jt)
        wait_scatter(jt, rows_b, ss_b)

        pltpu.sync_copy(hist_v, cnt_sh.at[ident_v.at[0]], add=True)

        plsc.subcore_barrier()

        w0 = s * nw_rows
        pltpu.sync_copy(acc_sh.at[pl.ds(w0, nw_rows)],
                        sum_hbm.at[pl.ds(c * half + w0, nw_rows)])

        @pl.when(s < cr // 8)
        def _():
            pltpu.sync_copy(cnt_sh.at[pl.ds(s * 8, 8)],
                            cnt_hbm.at[c, pl.ds(s * 8, 8)])

    return body(x_pad, edges3, ident_in)


def _tc_combine(ssum, cnt, x_pad, W_l, b_l2, W_r, n_pad, d, h):
    blk = n_pad // 8

    def body(p_ref, c_ref, x_ref, wl_ref, bl_ref, wr_ref, o_ref):
        cnt = c_ref[0, 0]
        inv = 1.0 / jnp.clip(cnt, 1.0, None)
        mean = p_ref[...] * inv[:, None]
        o_ref[...] = (
            jnp.dot(mean, wl_ref[...], preferred_element_type=jnp.float32)
            + bl_ref[...]
            + jnp.dot(x_ref[...], wr_ref[...], preferred_element_type=jnp.float32)
        )

    return pl.pallas_call(
        body,
        grid=(n_pad // blk,),
        in_specs=[
            pl.BlockSpec((blk, d), lambda i: (i, 0)),
            pl.BlockSpec((1, 1, blk), lambda i: (i, 0, 0)),
            pl.BlockSpec((blk, d), lambda i: (i, 0)),
            pl.BlockSpec((d, h), lambda i: (0, 0)),
            pl.BlockSpec((1, h), lambda i: (0, 0)),
            pl.BlockSpec((d, h), lambda i: (0, 0)),
        ],
        out_specs=pl.BlockSpec((blk, h), lambda i: (i, 0)),
        out_shape=jax.ShapeDtypeStruct((n_pad, h), jnp.float32),
    )(ssum, cnt, x_pad, W_l, b_l2, W_r)


def kernel(x, edge_index, W_l, b_l, W_r):
    n, d = x.shape
    h = W_l.shape[1]
    e = edge_index.shape[1]

    n_pad = ((n + 255) // 256) * 256
    per_tile = e // NS
    k = 128
    assert e % NS == 0
    nchunk = -(-per_tile // k)
    nchunk += nchunk % 2

    src_i = edge_index[0].astype(jnp.int32)
    dst_i = edge_index[1].astype(jnp.int32)
    dummy = jnp.int32(16383 << 14)
    packed = (src_i | (dst_i << 14)).reshape(NS, per_tile)
    packed = jnp.pad(packed, ((0, 0), (0, nchunk * k - per_tile)),
                     constant_values=dummy)
    edges3 = packed.reshape(NS, nchunk, k)
    x_pad = jnp.pad(x, ((0, n_pad - n), (0, 0)))

    cr = (n_pad // NC) // 128
    hr = ((cr + 1 + 7) // 8) * 8
    ident_in = jnp.arange(hr, dtype=jnp.int32).reshape(1, hr)

    ssum, cnt = _sc_aggregate(x_pad, edges3, ident_in,
                              n_pad, d, k, nchunk)
    blk = n_pad // 8
    cnt3 = cnt.reshape(n_pad // blk, 1, blk)
    out = _tc_combine(ssum, cnt3, x_pad, W_l, b_l.reshape(1, h), W_r,
                      n_pad, d, h)
    return out[:n]


# speedup vs baseline: 1.2072x; 1.2072x over previous
import functools

import jax
import jax.numpy as jnp
from jax import lax
from jax.experimental import pallas as pl
from jax.experimental.pallas import tpu as pltpu
from jax.experimental.pallas import tpu_sc as plsc

NC = 2
NS = 16
NW = NC * NS


def _sc_aggregate(x_pad, src3, dst3, zrows_in,
                  n_pad, d, k, nchunk):
    half = n_pad // NC
    acc_rows = half + 8
    nz = half // NS
    nw_rows = half // NS
    mesh = plsc.VectorSubcoreMesh(core_axis_name="c", subcore_axis_name="s")

    @functools.partial(
        pl.kernel,
        mesh=mesh,
        compiler_params=pltpu.CompilerParams(needs_layout_passes=False),
        out_type=[
            jax.ShapeDtypeStruct((n_pad, d), jnp.float32),
            jax.ShapeDtypeStruct((NC, NS, half), jnp.float32),
        ],
        scratch_types=[
            pltpu.VMEM((nchunk, k), jnp.int32),
            pltpu.VMEM((nchunk, k), jnp.int32),
            pltpu.VMEM((k, d), jnp.float32),
            pltpu.VMEM((8, d), jnp.float32),
            pltpu.VMEM((half + 16,), jnp.float32),
            pltpu.VMEM_SHARED((acc_rows, d), jnp.float32),
            pltpu.SemaphoreType.DMA,
        ],
    )
    def body(x_hbm, src_hbm, dst_hbm, zr_hbm,
             sum_hbm, cnt_hbm,
             src_v, dst_v, rows_v, zrow_v, hist_v,
             acc_sh, sem):
        c = lax.axis_index("c")
        s = lax.axis_index("s")

        pltpu.sync_copy(zr_hbm, zrow_v)

        zv = jnp.zeros((16,), jnp.float32)

        def zero_hist(i, carry):
            hist_v[pl.ds(i * 16, 16)] = zv
            return carry

        lax.fori_loop(0, (half + 16) // 16, zero_hist, 0)

        z0 = s * nz

        def zero_body(i, carry):
            pltpu.sync_copy(zrow_v, acc_sh.at[pl.ds(z0 + i * 8, 8)])
            return carry

        lax.fori_loop(0, nz // 8, zero_body, 0)

        @pl.when(s == NS - 1)
        def _():
            pltpu.sync_copy(zrow_v, acc_sh.at[pl.ds(half, 8)])

        pltpu.sync_copy(src_hbm.at[s], src_v)
        pltpu.sync_copy(dst_hbm.at[s], dst_v)

        lo = c * half
        hi = lo + half
        dump = jnp.full((16,), half, jnp.int32)

        def xform_body(r, carry):
            for q in range(k // 16):
                v = dst_v[r, pl.ds(q * 16, 16)]
                ok = (v >= lo) & (v < hi)
                dst_v[r, pl.ds(q * 16, 16)] = jnp.where(ok, v - lo, dump)
            return carry

        lax.fori_loop(0, nchunk, xform_body, 0)

        plsc.subcore_barrier()

        ones16 = jnp.ones((16,), jnp.float32)

        def edge_body(j, carry):
            pltpu.async_copy(x_hbm.at[src_v.at[j]], rows_v, sem).wait()
            pltpu.sync_copy(rows_v, acc_sh.at[dst_v.at[j]], add=True)
            for q in range(k // 16):
                idx = dst_v[j, pl.ds(q * 16, 16)]
                occ, last = plsc.scan_count(idx)
                old = plsc.load_gather(hist_v, [idx])
                plsc.store_scatter(hist_v, [idx],
                                   old + occ.astype(jnp.float32), mask=last)
            return carry

        lax.fori_loop(0, nchunk, edge_body, 0)

        plsc.subcore_barrier()

        w0 = s * nw_rows
        pltpu.sync_copy(acc_sh.at[pl.ds(w0, nw_rows)],
                        sum_hbm.at[pl.ds(c * half + w0, nw_rows)])
        pltpu.sync_copy(hist_v.at[pl.ds(0, half)], cnt_hbm.at[c, s])

    return body(x_pad, src3, dst3, zrows_in)


def _tc_combine(ssum, cnt, x_pad, W_l, b_l2, W_r, n_pad, d, h):
    blk = n_pad // 8
    half = n_pad // NC
    bph = half // blk

    def body(p_ref, c_ref, x_ref, wl_ref, bl_ref, wr_ref, o_ref):
        cnt = jnp.sum(c_ref[0], axis=0)
        inv = 1.0 / jnp.clip(cnt, 1.0, None)
        mean = p_ref[...] * inv[:, None]
        o_ref[...] = (
            jnp.dot(mean, wl_ref[...], preferred_element_type=jnp.float32)
            + bl_ref[...]
            + jnp.dot(x_ref[...], wr_ref[...], preferred_element_type=jnp.float32)
        )

    return pl.pallas_call(
        body,
        grid=(n_pad // blk,),
        in_specs=[
            pl.BlockSpec((blk, d), lambda i: (i, 0)),
            pl.BlockSpec((1, NS, blk), lambda i: (i // bph, 0, i % bph)),
            pl.BlockSpec((blk, d), lambda i: (i, 0)),
            pl.BlockSpec((d, h), lambda i: (0, 0)),
            pl.BlockSpec((1, h), lambda i: (0, 0)),
            pl.BlockSpec((d, h), lambda i: (0, 0)),
        ],
        out_specs=pl.BlockSpec((blk, h), lambda i: (i, 0)),
        out_shape=jax.ShapeDtypeStruct((n_pad, h), jnp.float32),
    )(ssum, cnt, x_pad, W_l, b_l2, W_r)


def kernel(x, edge_index, W_l, b_l, W_r):
    n, d = x.shape
    h = W_l.shape[1]
    e = edge_index.shape[1]

    n_pad = ((n + 255) // 256) * 256
    per_tile = e // NS
    k = 80
    assert e % NS == 0 and per_tile % k == 0
    nchunk = per_tile // k

    src3 = edge_index[0].astype(jnp.int32).reshape(NS, nchunk, k)
    dst3 = edge_index[1].astype(jnp.int32).reshape(NS, nchunk, k)
    x_pad = jnp.pad(x, ((0, n_pad - n), (0, 0)))

    zrows_in = jnp.zeros((8, d), jnp.float32)

    ssum, cnt = _sc_aggregate(x_pad, src3, dst3, zrows_in,
                              n_pad, d, k, nchunk)
    out = _tc_combine(ssum, cnt, x_pad, W_l, b_l.reshape(1, h), W_r,
                      n_pad, d, h)
    return out[:n]
